# baseline (speedup 1.0000x reference)
import functools

import jax
import jax.numpy as jnp
from jax import lax
from jax.experimental import pallas as pl
from jax.experimental.pallas import tpu as pltpu
from jax.experimental.pallas import tpu_sc as plsc

B = 4096
N_CAT = 26
N_CLASS = 100000
PER = 56
SH = 8
EMB = PER + SH

_INFO = plsc.get_sparse_core_info()
NC = _INFO.num_cores
NS = _INFO.num_subcores
NW = NC * NS
BPW = B // NW


def _body(xt, tab, sh_big, out, idx2, shv, g0, g1,
          sem_i, sem_g0, sem_g1, sem_o0, sem_o1, sem_o8):
    wid = lax.axis_index("s") * NC + lax.axis_index("c")
    b0 = pl.multiple_of(wid * BPW, BPW)

    c_idx = pltpu.async_copy(xt.at[:, pl.ds(b0, BPW)], idx2, sem_i)
    c_shv = pltpu.async_copy(sh_big, shv, sem_i)
    c_idx.wait()
    c_shv.wait()

    for i in range(1, N_CAT):
        off = jnp.full((16,), i * N_CLASS, dtype=jnp.int32)
        for k in range(BPW // 16):
            sl = pl.ds(k * 16, 16)
            idx2[i, sl] = idx2[i, sl] + off

    gbufs = (g0, g1)
    gsems = (sem_g0, sem_g1)
    osems = (sem_o0, sem_o1)
    gathers = [None, None]
    outs = [None, None]
    o8s = []

    gathers[0] = pltpu.async_copy(tab.at[idx2.at[0]], gbufs[0], gsems[0])
    for i in range(N_CAT):
        bsel = i % 2
        nsel = (i + 1) % 2
        gathers[bsel].wait()
        if i + 1 < N_CAT:
            if outs[nsel] is not None:
                outs[nsel].wait()
                outs[nsel] = None
            gathers[nsel] = pltpu.async_copy(
                tab.at[idx2.at[i + 1]], gbufs[nsel], gsems[nsel])
        outs[bsel] = pltpu.async_copy(
            gbufs[bsel], out.at[pl.ds(b0, BPW), i, pl.ds(0, PER)], osems[bsel])
        o8s.append(pltpu.async_copy(
            shv.at[i], out.at[pl.ds(b0, BPW), i, pl.ds(PER, SH)], sem_o8))

    for d in outs:
        if d is not None:
            d.wait()
    for d in o8s:
        d.wait()


_MESH = plsc.VectorSubcoreMesh(core_axis_name="c", subcore_axis_name="s")

_sc_emb = functools.partial(
    pl.kernel,
    mesh=_MESH,
    compiler_params=pltpu.CompilerParams(use_tc_tiling_on_sc=False),
    out_type=jax.ShapeDtypeStruct((B, N_CAT, EMB), jnp.float32),
    scratch_types=[
        pltpu.VMEM((N_CAT, BPW), jnp.int32),
        pltpu.VMEM((N_CAT, BPW, SH), jnp.float32),
        pltpu.VMEM((BPW, PER), jnp.float32),
        pltpu.VMEM((BPW, PER), jnp.float32),
        pltpu.SemaphoreType.DMA,
        pltpu.SemaphoreType.DMA,
        pltpu.SemaphoreType.DMA,
        pltpu.SemaphoreType.DMA,
        pltpu.SemaphoreType.DMA,
        pltpu.SemaphoreType.DMA,
    ],
)(_body)


def kernel(x, tables, shares):
    xt = x.T
    tab = tables.reshape(N_CAT * N_CLASS, PER)
    sh_big = jnp.broadcast_to(shares[:, None, :], (N_CAT, BPW, SH))
    return _sc_emb(xt, tab, sh_big)


# speedup vs baseline: 1.0142x; 1.0142x over previous
import functools

import jax
import jax.numpy as jnp
from jax import lax
from jax.experimental import pallas as pl
from jax.experimental.pallas import tpu as pltpu
from jax.experimental.pallas import tpu_sc as plsc

B = 4096
N_CAT = 26
N_CLASS = 100000
PER = 56
SH = 8
EMB = PER + SH

_INFO = plsc.get_sparse_core_info()
NC = _INFO.num_cores
NS = _INFO.num_subcores
NW = NC * NS
BPW = B // NW
CH = 32
NCHUNK = BPW // CH


def _body(xt, tab, tmpl, out, idx2, gall, shout,
          sem_i, sem_g, sem_l, sem_o):
    sid = lax.axis_index("s")
    wid = sid * NC + lax.axis_index("c")
    b0 = pl.multiple_of(wid * BPW, BPW)
    bigout = shout.at[sid]

    c_idx = pltpu.async_copy(xt.at[:, pl.ds(b0, BPW)], idx2, sem_i)
    c_t = pltpu.async_copy(tmpl, bigout, sem_i)
    c_idx.wait()
    c_t.wait()

    def fire(c):
        return [
            pltpu.async_copy(
                tab.at[idx2.at[i, pl.ds(c * CH, CH)]],
                gall.at[i],
                sem_g)
            for i in range(N_CAT)
        ]

    gathers = fire(0)
    out_dma = None
    for c in range(NCHUNK):
        for g in gathers:
            g.wait()
        if out_dma is not None:
            out_dma.wait()
        locs = [
            pltpu.async_copy(
                gall.at[i], bigout.at[:, pl.ds(i * EMB, PER)], sem_l)
            for i in range(N_CAT)
        ]
        for l in locs:
            l.wait()
        if c + 1 < NCHUNK:
            gathers = fire(c + 1)
        out_dma = pltpu.async_copy(
            bigout, out.at[pl.ds(b0 + c * CH, CH)], sem_o)
    out_dma.wait()


_MESH = plsc.VectorSubcoreMesh(core_axis_name="c", subcore_axis_name="s")

_sc_emb = functools.partial(
    pl.kernel,
    mesh=_MESH,
    compiler_params=pltpu.CompilerParams(use_tc_tiling_on_sc=False),
    out_type=jax.ShapeDtypeStruct((B, N_CAT * EMB), jnp.float32),
    scratch_types=[
        pltpu.VMEM((N_CAT, BPW), jnp.int32),
        pltpu.VMEM((N_CAT, CH, PER), jnp.float32),
        pltpu.VMEM_SHARED((NS, CH, N_CAT * EMB), jnp.float32),
        pltpu.SemaphoreType.DMA,
        pltpu.SemaphoreType.DMA,
        pltpu.SemaphoreType.DMA,
        pltpu.SemaphoreType.DMA,
    ],
)(_body)


def kernel(x, tables, shares):
    xt = x.T + (jnp.arange(N_CAT, dtype=jnp.int32) * N_CLASS)[:, None]
    tab = tables.reshape(N_CAT * N_CLASS, PER)
    tmpl = jnp.concatenate(
        [jnp.zeros((CH, N_CAT, PER), jnp.float32),
         jnp.broadcast_to(shares[None, :, :], (CH, N_CAT, SH))],
        axis=-1).reshape(CH, N_CAT * EMB)
    return _sc_emb(xt, tab, tmpl).reshape(B, N_CAT, EMB)
